# baseline (speedup 1.0000x reference)
import functools

import numpy as np
import jax
import jax.numpy as jnp
from jax import lax
from jax.experimental import pallas as pl
from jax.experimental.pallas import tpu as pltpu
from jax.experimental.pallas import tpu_sc as plsc

B, N, D_IN, D_HID = 512, 128, 768, 16
G = 4
NW = 32
GPW = B // NW
CPW = GPW * N

_RSQRT_TBL = np.concatenate([[1.0], 1.0 / np.sqrt(np.arange(1, 256))]).astype(
    np.float32)


def _tc_stage1(x_ref, w1_ref, ht_ref):
    h = jnp.dot(x_ref[0].astype(jnp.bfloat16), w1_ref[...].astype(jnp.bfloat16),
                preferred_element_type=jnp.float32)
    ht_ref[0] = h.T


def _sc_agg(ht_hbm, hd_hbm, tbl_hbm, b1_hbm, a2_hbm,
            h_v, out_v, u_v, acc_v, hd_v, deg_v, dinv_v, tbl_v, b1_v):
    wid = lax.axis_index("s") * 2 + lax.axis_index("c")
    pltpu.sync_copy(tbl_hbm, tbl_v)
    pltpu.sync_copy(b1_hbm, b1_v)
    pltpu.sync_copy(hd_hbm.at[wid], hd_v)
    pltpu.sync_copy(ht_hbm.at[wid], h_v)

    ones_i = jnp.ones((16,), jnp.int32)
    b1s = [b1_v[f, :] for f in range(D_HID)]
    NC = N // 16

    def graph_body(g, carry):
        col0 = g * N
        for c in range(NC):
            deg_v[pl.ds(c * 16, 16)] = jnp.zeros((16,), jnp.int32)
        hds = []
        for c in range(NC):
            hd_c = hd_v[pl.ds(col0 + c * 16, 16)]
            hds.append(hd_c)
            plsc.addupdate_scatter(deg_v, [hd_c], ones_i)
        for c in range(NC):
            d_c = deg_v[pl.ds(c * 16, 16)] + 1
            dinv_v[pl.ds(c * 16, 16)] = plsc.load_gather(tbl_v, [d_c])

        for f in range(D_HID):
            fidx = jnp.full((16,), f, jnp.int32)
            for c in range(NC):
                u_c = h_v[f, pl.ds(col0 + c * 16, 16)] * dinv_v[pl.ds(c * 16, 16)]
                u_v[f, pl.ds(c * 16, 16)] = u_c
                acc_v[f, pl.ds(c * 16, 16)] = u_c
            for c in range(NC):
                plsc.addupdate_scatter(
                    acc_v, [fidx, hds[c]], u_v[f, pl.ds(c * 16, 16)])
            for c in range(NC):
                dv = dinv_v[pl.ds(c * 16, 16)]
                h1 = jnp.maximum(acc_v[f, pl.ds(c * 16, 16)] * dv + b1s[f], 0.0)
                u2 = h1 * dv
                u_v[f, pl.ds(c * 16, 16)] = u2
                acc_v[f, pl.ds(c * 16, 16)] = u2
            for c in range(NC):
                plsc.addupdate_scatter(
                    acc_v, [fidx, hds[c]], u_v[f, pl.ds(c * 16, 16)])
            for c in range(NC):
                out_v[f, pl.ds(col0 + c * 16, 16)] = (
                    acc_v[f, pl.ds(c * 16, 16)] * dinv_v[pl.ds(c * 16, 16)])
        return carry

    lax.fori_loop(0, GPW, graph_body, 0)
    pltpu.sync_copy(out_v, a2_hbm.at[wid])


def _tc_stage2(a2_ref, w2_ref, out_ref):
    GN = G * N
    a2 = a2_ref[0].T
    m = jnp.dot(a2.astype(jnp.bfloat16), w2_ref[...].astype(jnp.bfloat16),
                preferred_element_type=jnp.float32)
    m3 = m.reshape(G, N, D_IN)
    mx = jnp.max(m3, axis=1, keepdims=True)
    lse = mx + jnp.log(jnp.sum(jnp.exp(m3 - mx), axis=1, keepdims=True))
    out_ref[0] = (m3 - lse).reshape(GN, D_IN)


@jax.jit
def kernel(head, x, W1, b1, W2, b2):
    del b2
    spg = G * N
    ht = pl.pallas_call(
        _tc_stage1,
        grid=(B // G,),
        in_specs=[
            pl.BlockSpec((1, spg, D_IN), lambda i: (i, 0, 0)),
            pl.BlockSpec((D_IN, D_HID), lambda i: (0, 0)),
        ],
        out_specs=pl.BlockSpec((1, D_HID, spg), lambda i: (i * G // GPW, 0, i % (GPW // G))),
        out_shape=jax.ShapeDtypeStruct((NW, D_HID, CPW), jnp.float32),
        compiler_params=pltpu.CompilerParams(
            dimension_semantics=("parallel",),
        ),
    )(x.reshape(B // G, spg, D_IN), W1)

    sc_call = functools.partial(
        pl.kernel,
        out_type=jax.ShapeDtypeStruct((NW, D_HID, CPW), jnp.float32),
        mesh=plsc.VectorSubcoreMesh(core_axis_name="c", subcore_axis_name="s"),
        compiler_params=pltpu.CompilerParams(needs_layout_passes=False),
        scratch_types=[
            pltpu.VMEM((D_HID, CPW), jnp.float32),
            pltpu.VMEM((D_HID, CPW), jnp.float32),
            pltpu.VMEM((D_HID, N), jnp.float32),
            pltpu.VMEM((D_HID, N), jnp.float32),
            pltpu.VMEM((CPW,), jnp.int32),
            pltpu.VMEM((N,), jnp.int32),
            pltpu.VMEM((N,), jnp.float32),
            pltpu.VMEM((256,), jnp.float32),
            pltpu.VMEM((D_HID, 16), jnp.float32),
        ],
    )(_sc_agg)
    b1bc = jnp.broadcast_to(b1.reshape(D_HID, 1), (D_HID, 16))
    a2t = sc_call(ht, head.reshape(NW, CPW), jnp.asarray(_RSQRT_TBL), b1bc)

    out = pl.pallas_call(
        _tc_stage2,
        grid=(B // G,),
        in_specs=[
            pl.BlockSpec((1, D_HID, spg), lambda i: (i * G // GPW, 0, i % (GPW // G))),
            pl.BlockSpec((D_HID, D_IN), lambda i: (0, 0)),
        ],
        out_specs=pl.BlockSpec((1, spg, D_IN), lambda i: (i, 0, 0)),
        out_shape=jax.ShapeDtypeStruct((B // G, spg, D_IN), jnp.float32),
        compiler_params=pltpu.CompilerParams(
            dimension_semantics=("parallel",),
        ),
    )(a2t, W2)
    return out.reshape(B, N, D_IN)


# speedup vs baseline: 1.2168x; 1.2168x over previous
import functools

import numpy as np
import jax
import jax.numpy as jnp
from jax import lax
from jax.experimental import pallas as pl
from jax.experimental.pallas import tpu as pltpu
from jax.experimental.pallas import tpu_sc as plsc

B, N, D_IN, D_HID = 512, 128, 768, 16
G = 4
NW = 32
NCHUNK = 2
BC = B // NCHUNK
GPW = BC // NW
CPW = GPW * N
SPG = G * N
SPW = CPW // SPG

_RSQRT_TBL = np.concatenate([[1.0], 1.0 / np.sqrt(np.arange(1, 256))]).astype(
    np.float32)


def _tc_stage1(x_ref, w1_ref, ht_ref):
    h = jnp.dot(x_ref[0].astype(jnp.bfloat16), w1_ref[...].astype(jnp.bfloat16),
                preferred_element_type=jnp.float32)
    ht_ref[0] = h.T


def _sc_agg(ht_hbm, hd_hbm, tbl_hbm, b1_hbm, a2_hbm,
            h_v, out_v, u_v, acc_v, hd_v, deg_v, dinv_v, tbl_v, b1_v):
    wid = lax.axis_index("s") * 2 + lax.axis_index("c")
    pltpu.sync_copy(tbl_hbm, tbl_v)
    pltpu.sync_copy(b1_hbm, b1_v)
    pltpu.sync_copy(hd_hbm.at[wid], hd_v)
    pltpu.sync_copy(ht_hbm.at[wid], h_v)

    ones_i = jnp.ones((16,), jnp.int32)
    b1s = [b1_v[f, :] for f in range(D_HID)]
    NC = N // 16

    def graph_body(g, carry):
        col0 = g * N
        for c in range(NC):
            deg_v[pl.ds(c * 16, 16)] = jnp.zeros((16,), jnp.int32)
        hds = []
        for c in range(NC):
            hd_c = hd_v[pl.ds(col0 + c * 16, 16)]
            hds.append(hd_c)
            plsc.addupdate_scatter(deg_v, [hd_c], ones_i)
        for c in range(NC):
            d_c = deg_v[pl.ds(c * 16, 16)] + 1
            dinv_v[pl.ds(c * 16, 16)] = plsc.load_gather(tbl_v, [d_c])

        for f in range(D_HID):
            fidx = jnp.full((16,), f, jnp.int32)
            for c in range(NC):
                u_c = h_v[f, pl.ds(col0 + c * 16, 16)] * dinv_v[pl.ds(c * 16, 16)]
                u_v[f, pl.ds(c * 16, 16)] = u_c
                acc_v[f, pl.ds(c * 16, 16)] = u_c
            for c in range(NC):
                plsc.addupdate_scatter(
                    acc_v, [fidx, hds[c]], u_v[f, pl.ds(c * 16, 16)])
            for c in range(NC):
                dv = dinv_v[pl.ds(c * 16, 16)]
                h1 = jnp.maximum(acc_v[f, pl.ds(c * 16, 16)] * dv + b1s[f], 0.0)
                u2 = h1 * dv
                u_v[f, pl.ds(c * 16, 16)] = u2
                acc_v[f, pl.ds(c * 16, 16)] = u2
            for c in range(NC):
                plsc.addupdate_scatter(
                    acc_v, [fidx, hds[c]], u_v[f, pl.ds(c * 16, 16)])
            for c in range(NC):
                out_v[f, pl.ds(col0 + c * 16, 16)] = (
                    acc_v[f, pl.ds(c * 16, 16)] * dinv_v[pl.ds(c * 16, 16)])
        return carry

    lax.fori_loop(0, GPW, graph_body, 0)
    pltpu.sync_copy(out_v, a2_hbm.at[wid])


def _tc_stage2(a2_ref, w2_ref, prev_ref, out_ref):
    del prev_ref
    a2 = a2_ref[0].T
    m = jnp.dot(a2.astype(jnp.bfloat16), w2_ref[...].astype(jnp.bfloat16),
                preferred_element_type=jnp.float32)
    m3 = m.reshape(G, N, D_IN)
    mx = jnp.max(m3, axis=1, keepdims=True)
    lse = mx + jnp.log(jnp.sum(jnp.exp(m3 - mx), axis=1, keepdims=True))
    out_ref[0] = (m3 - lse).reshape(SPG, D_IN)


@jax.jit
def kernel(head, x, W1, b1, W2, b2):
    del b2
    xb = x.reshape(B // G, SPG, D_IN)
    hdw = head.reshape(NCHUNK, NW, CPW)
    tbl = jnp.asarray(_RSQRT_TBL)
    b1bc = jnp.broadcast_to(b1.reshape(D_HID, 1), (D_HID, 16))
    steps = BC // G

    sc_call = functools.partial(
        pl.kernel,
        out_type=jax.ShapeDtypeStruct((NW, D_HID, CPW), jnp.float32),
        mesh=plsc.VectorSubcoreMesh(core_axis_name="c", subcore_axis_name="s"),
        compiler_params=pltpu.CompilerParams(needs_layout_passes=False),
        scratch_types=[
            pltpu.VMEM((D_HID, CPW), jnp.float32),
            pltpu.VMEM((D_HID, CPW), jnp.float32),
            pltpu.VMEM((D_HID, N), jnp.float32),
            pltpu.VMEM((D_HID, N), jnp.float32),
            pltpu.VMEM((CPW,), jnp.int32),
            pltpu.VMEM((N,), jnp.int32),
            pltpu.VMEM((N,), jnp.float32),
            pltpu.VMEM((256,), jnp.float32),
            pltpu.VMEM((D_HID, 16), jnp.float32),
        ],
    )(_sc_agg)

    a2ts = []
    for c in range(NCHUNK):
        ht = pl.pallas_call(
            _tc_stage1,
            grid=(steps,),
            in_specs=[
                pl.BlockSpec((1, SPG, D_IN), lambda i, c=c: (i + c * steps, 0, 0)),
                pl.BlockSpec((D_IN, D_HID), lambda i: (0, 0)),
            ],
            out_specs=pl.BlockSpec(
                (1, D_HID, SPG), lambda i: (i // SPW, 0, i % SPW)),
            out_shape=jax.ShapeDtypeStruct((NW, D_HID, CPW), jnp.float32),
            compiler_params=pltpu.CompilerParams(
                dimension_semantics=("parallel",),
            ),
        )(xb, W1)
        a2ts.append(sc_call(ht, hdw[c], tbl, b1bc))

    out = None
    for c in range(NCHUNK):
        kwargs = {}
        if c == 0:
            prev = jnp.zeros((8, 128), jnp.float32)
            prev_spec = pl.BlockSpec((8, 128), lambda i: (0, 0))
        else:
            prev = out
            prev_spec = pl.BlockSpec(memory_space=pl.ANY)
            kwargs["input_output_aliases"] = {2: 0}
        out = pl.pallas_call(
            _tc_stage2,
            grid=(steps,),
            in_specs=[
                pl.BlockSpec((1, D_HID, SPG), lambda i: (i // SPW, 0, i % SPW)),
                pl.BlockSpec((D_HID, D_IN), lambda i: (0, 0)),
                prev_spec,
            ],
            out_specs=pl.BlockSpec(
                (1, SPG, D_IN), lambda i, c=c: (i + c * steps, 0, 0)),
            out_shape=jax.ShapeDtypeStruct((B // G, SPG, D_IN), jnp.float32),
            compiler_params=pltpu.CompilerParams(
                dimension_semantics=("parallel",),
            ),
            **kwargs,
        )(a2ts[c], W2, prev)
    return out.reshape(B, N, D_IN)


# speedup vs baseline: 1.6339x; 1.3427x over previous
import functools

import numpy as np
import jax
import jax.numpy as jnp
from jax import lax
from jax.experimental import pallas as pl
from jax.experimental.pallas import tpu as pltpu
from jax.experimental.pallas import tpu_sc as plsc

B, N, D_IN, D_HID = 512, 128, 768, 16
G = 8
NW = 32
NCHUNK = 2
BC = B // NCHUNK
GPW = BC // NW
CPW = GPW * N
SPG = G * N
SPW = CPW // SPG

_RSQRT_TBL = np.concatenate([[1.0], 1.0 / np.sqrt(np.arange(1, 256))]).astype(
    np.float32)


def _tc_stage1(x_ref, w1_ref, ht_ref):
    h = jnp.dot(x_ref[0].astype(jnp.bfloat16), w1_ref[...].astype(jnp.bfloat16),
                preferred_element_type=jnp.float32)
    ht_ref[0] = h.T


def _sc_agg(ht_hbm, hd_hbm, tbl_hbm, b1_hbm, a2_hbm,
            h_v, out_v, u_v, acc_v, hd_v, deg_v, dinv_v, tbl_v, b1_v):
    wid = lax.axis_index("s") * 2 + lax.axis_index("c")
    pltpu.sync_copy(tbl_hbm, tbl_v)
    pltpu.sync_copy(b1_hbm, b1_v)
    pltpu.sync_copy(hd_hbm.at[wid], hd_v)
    pltpu.sync_copy(ht_hbm.at[wid], h_v)

    ones_i = jnp.ones((16,), jnp.int32)
    b1s = [b1_v[f, :] for f in range(D_HID)]
    NC = N // 16

    def graph_body(g, carry):
        col0 = g * N
        for c in range(NC):
            deg_v[pl.ds(c * 16, 16)] = jnp.zeros((16,), jnp.int32)
        hds = []
        for c in range(NC):
            hd_c = hd_v[pl.ds(col0 + c * 16, 16)]
            hds.append(hd_c)
            plsc.addupdate_scatter(deg_v, [hd_c], ones_i)
        for c in range(NC):
            d_c = deg_v[pl.ds(c * 16, 16)] + 1
            dinv_v[pl.ds(c * 16, 16)] = plsc.load_gather(tbl_v, [d_c])

        for f in range(D_HID):
            fidx = jnp.full((16,), f, jnp.int32)
            for c in range(NC):
                u_c = h_v[f, pl.ds(col0 + c * 16, 16)] * dinv_v[pl.ds(c * 16, 16)]
                u_v[f, pl.ds(c * 16, 16)] = u_c
                acc_v[f, pl.ds(c * 16, 16)] = u_c
            for c in range(NC):
                plsc.addupdate_scatter(
                    acc_v, [fidx, hds[c]], u_v[f, pl.ds(c * 16, 16)])
            for c in range(NC):
                dv = dinv_v[pl.ds(c * 16, 16)]
                h1 = jnp.maximum(acc_v[f, pl.ds(c * 16, 16)] * dv + b1s[f], 0.0)
                u2 = h1 * dv
                u_v[f, pl.ds(c * 16, 16)] = u2
                acc_v[f, pl.ds(c * 16, 16)] = u2
            for c in range(NC):
                plsc.addupdate_scatter(
                    acc_v, [fidx, hds[c]], u_v[f, pl.ds(c * 16, 16)])
            for c in range(NC):
                out_v[f, pl.ds(col0 + c * 16, 16)] = (
                    acc_v[f, pl.ds(c * 16, 16)] * dinv_v[pl.ds(c * 16, 16)])
        return carry

    lax.fori_loop(0, GPW, graph_body, 0)
    pltpu.sync_copy(out_v, a2_hbm.at[wid])


def _tc_stage2(a2_ref, w2_ref, prev_ref, out_ref):
    del prev_ref
    a2 = a2_ref[0].T
    m = jnp.dot(a2.astype(jnp.bfloat16), w2_ref[...].astype(jnp.bfloat16),
                preferred_element_type=jnp.float32)
    m3 = m.reshape(G, N, D_IN)
    mx = jnp.max(m3, axis=1, keepdims=True)
    lse = mx + jnp.log(jnp.sum(jnp.exp(m3 - mx), axis=1, keepdims=True))
    out_ref[0] = (m3 - lse).reshape(SPG, D_IN)


@jax.jit
def kernel(head, x, W1, b1, W2, b2):
    del b2
    xb = x.reshape(B // G, SPG, D_IN)
    hdw = head.reshape(NCHUNK, NW, CPW)
    tbl = jnp.asarray(_RSQRT_TBL)
    b1bc = jnp.broadcast_to(b1.reshape(D_HID, 1), (D_HID, 16))
    steps = BC // G

    sc_call = functools.partial(
        pl.kernel,
        out_type=jax.ShapeDtypeStruct((NW, D_HID, CPW), jnp.float32),
        mesh=plsc.VectorSubcoreMesh(core_axis_name="c", subcore_axis_name="s"),
        compiler_params=pltpu.CompilerParams(needs_layout_passes=False),
        scratch_types=[
            pltpu.VMEM((D_HID, CPW), jnp.float32),
            pltpu.VMEM((D_HID, CPW), jnp.float32),
            pltpu.VMEM((D_HID, N), jnp.float32),
            pltpu.VMEM((D_HID, N), jnp.float32),
            pltpu.VMEM((CPW,), jnp.int32),
            pltpu.VMEM((N,), jnp.int32),
            pltpu.VMEM((N,), jnp.float32),
            pltpu.VMEM((256,), jnp.float32),
            pltpu.VMEM((D_HID, 16), jnp.float32),
        ],
    )(_sc_agg)

    a2ts = []
    for c in range(NCHUNK):
        ht = pl.pallas_call(
            _tc_stage1,
            grid=(steps,),
            in_specs=[
                pl.BlockSpec((1, SPG, D_IN), lambda i, c=c: (i + c * steps, 0, 0)),
                pl.BlockSpec((D_IN, D_HID), lambda i: (0, 0)),
            ],
            out_specs=pl.BlockSpec(
                (1, D_HID, SPG), lambda i: (i // SPW, 0, i % SPW)),
            out_shape=jax.ShapeDtypeStruct((NW, D_HID, CPW), jnp.float32),
            compiler_params=pltpu.CompilerParams(
                dimension_semantics=("parallel",),
            ),
        )(xb, W1)
        a2ts.append(sc_call(ht, hdw[c], tbl, b1bc))

    out = None
    for c in range(NCHUNK):
        kwargs = {}
        if c == 0:
            prev = jnp.zeros((8, 128), jnp.float32)
            prev_spec = pl.BlockSpec((8, 128), lambda i: (0, 0))
        else:
            prev = out
            prev_spec = pl.BlockSpec(memory_space=pl.ANY)
            kwargs["input_output_aliases"] = {2: 0}
        out = pl.pallas_call(
            _tc_stage2,
            grid=(steps,),
            in_specs=[
                pl.BlockSpec((1, D_HID, SPG), lambda i: (i // SPW, 0, i % SPW)),
                pl.BlockSpec((D_HID, D_IN), lambda i: (0, 0)),
                prev_spec,
            ],
            out_specs=pl.BlockSpec(
                (1, SPG, D_IN), lambda i, c=c: (i + c * steps, 0, 0)),
            out_shape=jax.ShapeDtypeStruct((B // G, SPG, D_IN), jnp.float32),
            compiler_params=pltpu.CompilerParams(
                dimension_semantics=("parallel",),
            ),
            **kwargs,
        )(a2ts[c], W2, prev)
    return out.reshape(B, N, D_IN)


# speedup vs baseline: 1.7890x; 1.0949x over previous
import functools

import numpy as np
import jax
import jax.numpy as jnp
from jax import lax
from jax.experimental import pallas as pl
from jax.experimental.pallas import tpu as pltpu
from jax.experimental.pallas import tpu_sc as plsc

B, N, D_IN, D_HID = 512, 128, 768, 16
G = 16
NW = 32
NCHUNK = 2
BC = B // NCHUNK
GPW = BC // NW
CPW = GPW * N
SPG = G * N
WPS = SPG // CPW

_RSQRT_TBL = np.concatenate([[1.0], 1.0 / np.sqrt(np.arange(1, 256))]).astype(
    np.float32)


def _tc_stage1(x_ref, w1_ref, ht_ref):
    h = jnp.dot(x_ref[0].astype(jnp.bfloat16), w1_ref[...].astype(jnp.bfloat16),
                preferred_element_type=jnp.float32)
    ht = h.T
    for w in range(WPS):
        ht_ref[w] = jax.lax.slice(ht, (0, w * CPW), (D_HID, (w + 1) * CPW))


def _sc_agg(ht_hbm, hd_hbm, tbl_hbm, b1_hbm, a2_hbm,
            h_v, out_v, u_v, acc_v, hd_v, deg_v, dinv_v, tbl_v, b1_v):
    wid = lax.axis_index("s") * 2 + lax.axis_index("c")
    pltpu.sync_copy(tbl_hbm, tbl_v)
    pltpu.sync_copy(b1_hbm, b1_v)
    pltpu.sync_copy(hd_hbm.at[wid], hd_v)
    pltpu.sync_copy(ht_hbm.at[wid], h_v)

    ones_i = jnp.ones((16,), jnp.int32)
    b1s = [b1_v[f, :] for f in range(D_HID)]
    NC = N // 16

    def graph_body(g, carry):
        col0 = g * N
        for c in range(NC):
            deg_v[pl.ds(c * 16, 16)] = jnp.zeros((16,), jnp.int32)
        hds = []
        for c in range(NC):
            hd_c = hd_v[pl.ds(col0 + c * 16, 16)]
            hds.append(hd_c)
            plsc.addupdate_scatter(deg_v, [hd_c], ones_i)
        for c in range(NC):
            d_c = deg_v[pl.ds(c * 16, 16)] + 1
            dinv_v[pl.ds(c * 16, 16)] = plsc.load_gather(tbl_v, [d_c])

        for f in range(D_HID):
            fidx = jnp.full((16,), f, jnp.int32)
            for c in range(NC):
                u_c = h_v[f, pl.ds(col0 + c * 16, 16)] * dinv_v[pl.ds(c * 16, 16)]
                u_v[f, pl.ds(c * 16, 16)] = u_c
                acc_v[f, pl.ds(c * 16, 16)] = u_c
            for c in range(NC):
                plsc.addupdate_scatter(
                    acc_v, [fidx, hds[c]], u_v[f, pl.ds(c * 16, 16)])
            for c in range(NC):
                dv = dinv_v[pl.ds(c * 16, 16)]
                h1 = jnp.maximum(acc_v[f, pl.ds(c * 16, 16)] * dv + b1s[f], 0.0)
                u2 = h1 * dv
                u_v[f, pl.ds(c * 16, 16)] = u2
                acc_v[f, pl.ds(c * 16, 16)] = u2
            for c in range(NC):
                plsc.addupdate_scatter(
                    acc_v, [fidx, hds[c]], u_v[f, pl.ds(c * 16, 16)])
            for c in range(NC):
                out_v[f, pl.ds(col0 + c * 16, 16)] = (
                    acc_v[f, pl.ds(c * 16, 16)] * dinv_v[pl.ds(c * 16, 16)])
        return carry

    lax.fori_loop(0, GPW, graph_body, 0)
    pltpu.sync_copy(out_v, a2_hbm.at[wid])


def _tc_stage2(a2_ref, w2_ref, prev_ref, out_ref):
    del prev_ref
    a2t = jnp.concatenate([a2_ref[w] for w in range(WPS)], axis=1)
    a2 = a2t.T
    m = jnp.dot(a2.astype(jnp.bfloat16), w2_ref[...].astype(jnp.bfloat16),
                preferred_element_type=jnp.float32)
    m3 = m.reshape(G, N, D_IN)
    mx = jnp.max(m3, axis=1, keepdims=True)
    lse = mx + jnp.log(jnp.sum(jnp.exp(m3 - mx), axis=1, keepdims=True))
    out_ref[0] = (m3 - lse).reshape(SPG, D_IN)


@jax.jit
def kernel(head, x, W1, b1, W2, b2):
    del b2
    xb = x.reshape(B // G, SPG, D_IN)
    hdw = head.reshape(NCHUNK, NW, CPW)
    tbl = jnp.asarray(_RSQRT_TBL)
    b1bc = jnp.broadcast_to(b1.reshape(D_HID, 1), (D_HID, 16))
    steps = BC // G

    sc_call = functools.partial(
        pl.kernel,
        out_type=jax.ShapeDtypeStruct((NW, D_HID, CPW), jnp.float32),
        mesh=plsc.VectorSubcoreMesh(core_axis_name="c", subcore_axis_name="s"),
        compiler_params=pltpu.CompilerParams(needs_layout_passes=False),
        scratch_types=[
            pltpu.VMEM((D_HID, CPW), jnp.float32),
            pltpu.VMEM((D_HID, CPW), jnp.float32),
            pltpu.VMEM((D_HID, N), jnp.float32),
            pltpu.VMEM((D_HID, N), jnp.float32),
            pltpu.VMEM((CPW,), jnp.int32),
            pltpu.VMEM((N,), jnp.int32),
            pltpu.VMEM((N,), jnp.float32),
            pltpu.VMEM((256,), jnp.float32),
            pltpu.VMEM((D_HID, 16), jnp.float32),
        ],
    )(_sc_agg)

    a2ts = []
    for c in range(NCHUNK):
        ht = pl.pallas_call(
            _tc_stage1,
            grid=(steps,),
            in_specs=[
                pl.BlockSpec((1, SPG, D_IN), lambda i, c=c: (i + c * steps, 0, 0)),
                pl.BlockSpec((D_IN, D_HID), lambda i: (0, 0)),
            ],
            out_specs=pl.BlockSpec(
                (WPS, D_HID, CPW), lambda i: (i, 0, 0)),
            out_shape=jax.ShapeDtypeStruct((NW, D_HID, CPW), jnp.float32),
            compiler_params=pltpu.CompilerParams(
                dimension_semantics=("parallel",),
            ),
        )(xb, W1)
        a2ts.append(sc_call(ht, hdw[c], tbl, b1bc))

    out = None
    for c in range(NCHUNK):
        kwargs = {}
        if c == 0:
            prev = jnp.zeros((8, 128), jnp.float32)
            prev_spec = pl.BlockSpec((8, 128), lambda i: (0, 0))
        else:
            prev = out
            prev_spec = pl.BlockSpec(memory_space=pl.ANY)
            kwargs["input_output_aliases"] = {2: 0}
        out = pl.pallas_call(
            _tc_stage2,
            grid=(steps,),
            in_specs=[
                pl.BlockSpec((WPS, D_HID, CPW), lambda i: (i, 0, 0)),
                pl.BlockSpec((D_HID, D_IN), lambda i: (0, 0)),
                prev_spec,
            ],
            out_specs=pl.BlockSpec(
                (1, SPG, D_IN), lambda i, c=c: (i + c * steps, 0, 0)),
            out_shape=jax.ShapeDtypeStruct((B // G, SPG, D_IN), jnp.float32),
            compiler_params=pltpu.CompilerParams(
                dimension_semantics=("parallel",),
            ),
            **kwargs,
        )(a2ts[c], W2, prev)
    return out.reshape(B, N, D_IN)
